# baseline (speedup 1.0000x reference)
import functools

import jax
import jax.numpy as jnp
from jax import lax
from jax.experimental import pallas as pl
from jax.experimental.pallas import tpu as pltpu
from jax.experimental.pallas import tpu_sc as plsc

N_SPARSE_F = 26
VOCAB_SIZE = 100000
EMB_DIM = 16
N_SE = N_SPARSE_F * EMB_DIM
HALF_F = N_SPARSE_F // 2
HALF_SE = HALF_F * EMB_DIM

SC_CORES = 2
SC_SUBCORES = 16
SC_WORKERS = SC_CORES * SC_SUBCORES

CHUNK_B = 512
N_CHUNKS = 32
N_ITEMS = HALF_F * N_CHUNKS
ITEMS_PER_WORKER = N_ITEMS // SC_WORKERS


def _sc_gather_half(flat_half, idx_fm, batch):
    mesh = plsc.VectorSubcoreMesh(core_axis_name="c", subcore_axis_name="s")

    @functools.partial(
        pl.kernel,
        out_type=jax.ShapeDtypeStruct((batch, HALF_SE), jnp.float32),
        mesh=mesh,
        compiler_params=pltpu.CompilerParams(use_tc_tiling_on_sc=False),
        scratch_types=[
            pltpu.VMEM((CHUNK_B,), jnp.int32),
            pltpu.VMEM((CHUNK_B, EMB_DIM), jnp.float32),
            pltpu.SemaphoreType.DMA,
        ],
    )
    def gather_kernel(table_hbm, idx_hbm, out_hbm, idx_v, rows_v, sem):
        wid = lax.axis_index("s") * SC_CORES + lax.axis_index("c")

        @pl.loop(0, ITEMS_PER_WORKER)
        def _(j):
            item = wid * ITEMS_PER_WORKER + j
            f = item // N_CHUNKS
            c = item - f * N_CHUNKS
            b0 = pl.multiple_of(c * CHUNK_B, 8)
            i0 = pl.multiple_of(f * batch + b0, 8)
            pltpu.sync_copy(idx_hbm.at[pl.ds(i0, CHUNK_B)], idx_v)
            pltpu.async_copy(table_hbm.at[idx_v], rows_v, sem).wait()
            pltpu.sync_copy(rows_v,
                            out_hbm.at[pl.ds(b0, CHUNK_B),
                                       pl.ds(f * EMB_DIM, EMB_DIM)])

    return gather_kernel(flat_half, idx_fm)


def _mlp_body(seA_ref, seB_ref, df_ref, Wd_ref, bd_ref, W1a_ref, W1b_ref,
              W1d_ref, b1_ref, W2_ref, b2_ref, W3_ref, b3_ref, Wo_ref,
              bo_ref, out_ref):
    seA = seA_ref[...]
    seB = seB_ref[...]
    de = jnp.dot(df_ref[...], Wd_ref[...],
                 preferred_element_type=jnp.float32) + bd_ref[...]
    h = jnp.maximum(
        jnp.dot(seA, W1a_ref[...], preferred_element_type=jnp.float32)
        + jnp.dot(seB, W1b_ref[...], preferred_element_type=jnp.float32)
        + jnp.dot(de, W1d_ref[...], preferred_element_type=jnp.float32)
        + b1_ref[...], 0.0)
    h = jnp.maximum(
        jnp.dot(h, W2_ref[...], preferred_element_type=jnp.float32)
        + b2_ref[...], 0.0)
    h = jnp.maximum(
        jnp.dot(h, W3_ref[...], preferred_element_type=jnp.float32)
        + b3_ref[...], 0.0)
    fm = jnp.sum(seA, axis=1) + jnp.sum(seB, axis=1) + jnp.sum(de, axis=1)
    logit = jnp.dot(h, Wo_ref[...], preferred_element_type=jnp.float32)[:, 0]
    out_ref[...] = logit + bo_ref[...] + 0.1 * fm


def _mlp(seA, seB, df, Wd, bd, W1a, W1b, W1d, b1, W2, b2, W3, b3, Wo, bo,
         tile_b=2048):
    B = df.shape[0]

    def full(a):
        return pl.BlockSpec(a.shape, lambda i: tuple(0 for _ in a.shape))

    return pl.pallas_call(
        _mlp_body,
        grid=(B // tile_b,),
        in_specs=[
            pl.BlockSpec((tile_b, HALF_SE), lambda i: (i, 0)),
            pl.BlockSpec((tile_b, HALF_SE), lambda i: (i, 0)),
            pl.BlockSpec((tile_b, df.shape[1]), lambda i: (i, 0)),
            full(Wd), full(bd), full(W1a), full(W1b), full(W1d), full(b1),
            full(W2), full(b2), full(W3), full(b3), full(Wo), full(bo),
        ],
        out_specs=pl.BlockSpec((tile_b,), lambda i: (i,)),
        out_shape=jax.ShapeDtypeStruct((B,), jnp.float32),
    )(seA, seB, df, Wd, bd, W1a, W1b, W1d, b1, W2, b2, W3, b3, Wo, bo)


def kernel(sparse_features, dense_features, tables, Wd, bd, W1, b1, W2, b2,
           W3, b3, Wo, bo):
    B = sparse_features.shape[0]
    idx_t = sparse_features.astype(jnp.int32).T
    offs = jnp.arange(HALF_F, dtype=jnp.int32) * VOCAB_SIZE

    ses = []
    for h in range(2):
        flat_half = tables[h * HALF_F:(h + 1) * HALF_F].reshape(
            HALF_F * VOCAB_SIZE, EMB_DIM)
        idx_h = (idx_t[h * HALF_F:(h + 1) * HALF_F]
                 + offs[:, None]).reshape(-1)
        ses.append(_sc_gather_half(flat_half, idx_h, B))

    W1a = W1[:HALF_SE]
    W1b = W1[HALF_SE:N_SE]
    W1d = W1[N_SE:]
    return _mlp(ses[0], ses[1], dense_features, Wd, bd, W1a, W1b, W1d, b1,
                W2, b2, W3, b3, Wo, bo)


# speedup vs baseline: 1.4478x; 1.4478x over previous
import functools

import jax
import jax.numpy as jnp
from jax import lax
from jax.experimental import pallas as pl
from jax.experimental.pallas import tpu as pltpu
from jax.experimental.pallas import tpu_sc as plsc

N_SPARSE_F = 26
VOCAB_SIZE = 100000
EMB_DIM = 16
N_SE = N_SPARSE_F * EMB_DIM

SC_CORES = 2
SC_SUBCORES = 16
SC_WORKERS = SC_CORES * SC_SUBCORES

CHUNK_B = 1024
N_CHUNKS = 16
N_ITEMS = N_SPARSE_F * N_CHUNKS
ITEMS_PER_WORKER = N_ITEMS // SC_WORKERS


def _sc_gather(tables, idx_fm, batch):
    mesh = plsc.VectorSubcoreMesh(core_axis_name="c", subcore_axis_name="s")

    @functools.partial(
        pl.kernel,
        out_type=jax.ShapeDtypeStruct((batch, N_SE), jnp.float32),
        mesh=mesh,
        compiler_params=pltpu.CompilerParams(use_tc_tiling_on_sc=False),
        scratch_types=[
            pltpu.VMEM((CHUNK_B,), jnp.int32),
            pltpu.VMEM((CHUNK_B, EMB_DIM), jnp.float32),
            pltpu.SemaphoreType.DMA,
        ],
    )
    def gather_kernel(table_hbm, idx_hbm, out_hbm, idx_v, rows_v, sem):
        wid = lax.axis_index("s") * SC_CORES + lax.axis_index("c")

        @pl.loop(0, ITEMS_PER_WORKER)
        def _(j):
            item = wid * ITEMS_PER_WORKER + j
            f = item // N_CHUNKS
            c = item - f * N_CHUNKS
            b0 = pl.multiple_of(c * CHUNK_B, 8)
            i0 = pl.multiple_of(f * batch + b0, 8)
            pltpu.sync_copy(idx_hbm.at[pl.ds(i0, CHUNK_B)], idx_v)
            pltpu.async_copy(table_hbm.at[idx_v], rows_v, sem).wait()
            pltpu.sync_copy(rows_v,
                            out_hbm.at[pl.ds(b0, CHUNK_B),
                                       pl.ds(f * EMB_DIM, EMB_DIM)])

    return gather_kernel(tables, idx_fm)


def _mlp_body(se_ref, df_ref, Wd_ref, bd_ref, W1s_ref, W1d_ref, b1_ref,
              W2_ref, b2_ref, W3_ref, b3_ref, Wo_ref, bo_ref, out_ref):
    se = se_ref[...]
    de = jnp.dot(df_ref[...], Wd_ref[...],
                 preferred_element_type=jnp.float32) + bd_ref[...]
    h = jnp.maximum(
        jnp.dot(se, W1s_ref[...], preferred_element_type=jnp.float32)
        + jnp.dot(de, W1d_ref[...], preferred_element_type=jnp.float32)
        + b1_ref[...], 0.0)
    h = jnp.maximum(
        jnp.dot(h, W2_ref[...], preferred_element_type=jnp.float32)
        + b2_ref[...], 0.0)
    h = jnp.maximum(
        jnp.dot(h, W3_ref[...], preferred_element_type=jnp.float32)
        + b3_ref[...], 0.0)
    fm = jnp.sum(se, axis=1) + jnp.sum(de, axis=1)
    logit = jnp.dot(h, Wo_ref[...], preferred_element_type=jnp.float32)[:, 0]
    out_ref[...] = logit + bo_ref[...] + 0.1 * fm


def _mlp(se, df, Wd, bd, W1s, W1d, b1, W2, b2, W3, b3, Wo, bo, tile_b=2048):
    B = df.shape[0]

    def full(a):
        return pl.BlockSpec(a.shape, lambda i: tuple(0 for _ in a.shape))

    return pl.pallas_call(
        _mlp_body,
        grid=(B // tile_b,),
        in_specs=[
            pl.BlockSpec((tile_b, N_SE), lambda i: (i, 0)),
            pl.BlockSpec((tile_b, df.shape[1]), lambda i: (i, 0)),
            full(Wd), full(bd), full(W1s), full(W1d), full(b1),
            full(W2), full(b2), full(W3), full(b3), full(Wo), full(bo),
        ],
        out_specs=pl.BlockSpec((tile_b,), lambda i: (i,)),
        out_shape=jax.ShapeDtypeStruct((B,), jnp.float32),
    )(se, df, Wd, bd, W1s, W1d, b1, W2, b2, W3, b3, Wo, bo)


def kernel(sparse_features, dense_features, tables, Wd, bd, W1, b1, W2, b2,
           W3, b3, Wo, bo):
    B = sparse_features.shape[0]
    offs = jnp.arange(N_SPARSE_F, dtype=jnp.int32) * VOCAB_SIZE
    idx_fm = (sparse_features.astype(jnp.int32).T
              + offs[:, None]).reshape(-1)

    flat_tables = tables.reshape(N_SPARSE_F * VOCAB_SIZE, EMB_DIM)

    se = _sc_gather(flat_tables, idx_fm, B)

    W1s = W1[:N_SE]
    W1d = W1[N_SE:]
    return _mlp(se, dense_features, Wd, bd, W1s, W1d, b1, W2, b2, W3, b3,
                Wo, bo)
